# Initial kernel scaffold; baseline (speedup 1.0000x reference)
import jax
import jax.numpy as jnp
from jax.experimental import pallas as pl


def kernel(input, batch, num_graphs):
    raise NotImplementedError("write your pallas kernel here")


# speedup vs baseline: 4.2901x; 4.2901x over previous
import functools

import jax
import jax.numpy as jnp
from jax import lax
from jax.experimental import pallas as pl
from jax.experimental.pallas import tpu as pltpu
from jax.experimental.pallas import tpu_sc as plsc

_N = 100000
_D = 128
_G = 512
_NC = 2
_NS = 16
_NW = _NC * _NS
_CHUNK = 128
_FULL = _N // _CHUNK
_TAIL = _N - _FULL * _CHUNK
_JFULL = _FULL // _NW
_REM = _FULL - _JFULL * _NW
_ROUNDS = _JFULL + 1
_ACC_ROWS = 528
_ZROWS = _ACC_ROWS // _NS
_DUMMY = _G

_mesh = plsc.VectorSubcoreMesh(core_axis_name="c", subcore_axis_name="s")


@functools.partial(
    pl.kernel,
    mesh=_mesh,
    out_type=jax.ShapeDtypeStruct((_NC, _G, _D), jnp.float32),
    scratch_types=[
        pltpu.VMEM((_ROUNDS, _CHUNK), jnp.int32),
        pltpu.VMEM((_CHUNK, _D), jnp.float32),
        pltpu.VMEM((_ZROWS, _D), jnp.float32),
        pltpu.VMEM_SHARED((_ACC_ROWS, _D), jnp.float32),
    ],
)
def _sc_segment_sum(x_hbm, b_hbm, out_hbm, idx_v, rows_v, zbuf, acc_sh):
    c = lax.axis_index("c")
    s = lax.axis_index("s")
    w = c * _NS + s

    zero = jnp.zeros((16,), jnp.float32)
    for r in range(_ZROWS):
        for k in range(_D // 16):
            zbuf[r, pl.ds(k * 16, 16)] = zero
    pltpu.sync_copy(zbuf, acc_sh.at[pl.ds(s * _ZROWS, _ZROWS)])
    plsc.subcore_barrier()

    for j in range(_JFULL):
        off = (j * _NW + w) * _CHUNK
        pltpu.sync_copy(b_hbm.at[pl.ds(off, _CHUNK)], idx_v.at[j])
        pltpu.sync_copy(x_hbm.at[pl.ds(off, _CHUNK)], rows_v)
        pltpu.sync_copy(rows_v, acc_sh.at[idx_v.at[j]], add=True)

    @pl.when(w < _REM)
    def _leftover():
        off = (_JFULL * _NW + w) * _CHUNK
        pltpu.sync_copy(b_hbm.at[pl.ds(off, _CHUNK)], idx_v.at[_JFULL])
        pltpu.sync_copy(x_hbm.at[pl.ds(off, _CHUNK)], rows_v)
        pltpu.sync_copy(rows_v, acc_sh.at[idx_v.at[_JFULL]], add=True)

    @pl.when(w == _REM)
    def _tail():
        off = _FULL * _CHUNK
        dummy = jnp.full((16,), _DUMMY, jnp.int32)
        for k in range(_TAIL // 16, _CHUNK // 16):
            idx_v[_JFULL, pl.ds(k * 16, 16)] = dummy
        pltpu.sync_copy(b_hbm.at[pl.ds(off, _TAIL)],
                        idx_v.at[_JFULL, pl.ds(0, _TAIL)])
        pltpu.sync_copy(x_hbm.at[pl.ds(off, _TAIL)], rows_v.at[pl.ds(0, _TAIL)])
        pltpu.sync_copy(rows_v, acc_sh.at[idx_v.at[_JFULL]], add=True)

    plsc.subcore_barrier()

    pltpu.sync_copy(acc_sh.at[pl.ds(s * 32, 32)], rows_v.at[pl.ds(0, 32)])
    pltpu.sync_copy(rows_v.at[pl.ds(0, 32)], out_hbm.at[c, pl.ds(s * 32, 32)])


def _tc_combine(bias_ref, p_ref, o_ref):
    o_ref[...] = p_ref[0] + p_ref[1] + bias_ref[0]


def kernel(input, batch, num_graphs):
    x = input.astype(jnp.float32)
    b = batch.astype(jnp.int32)
    partials = _sc_segment_sum(x, b)
    bias = (jnp.asarray(num_graphs) - _G).astype(jnp.float32).reshape(1)
    out = pl.pallas_call(
        _tc_combine,
        out_shape=jax.ShapeDtypeStruct((_G, _D), jnp.float32),
        in_specs=[
            pl.BlockSpec(memory_space=pltpu.SMEM),
            pl.BlockSpec(memory_space=pltpu.VMEM),
        ],
    )(bias, partials)
    return out


# speedup vs baseline: 6.5768x; 1.5330x over previous
import functools

import jax
import jax.numpy as jnp
from jax import lax
from jax.experimental import pallas as pl
from jax.experimental.pallas import tpu as pltpu
from jax.experimental.pallas import tpu_sc as plsc

_N = 100000
_D = 128
_G = 512
_NC = 2
_NS = 16
_NW = _NC * _NS
_CHUNK = 128
_FULL = _N // _CHUNK
_TAIL = _N - _FULL * _CHUNK
_JFULL = _FULL // _NW
_REM = _FULL - _JFULL * _NW
_ROUNDS = _JFULL + 1
_ACC_ROWS = 528
_ZROWS = _ACC_ROWS // _NS
_DUMMY = _G

_mesh = plsc.VectorSubcoreMesh(core_axis_name="c", subcore_axis_name="s")


@functools.partial(
    pl.kernel,
    mesh=_mesh,
    out_type=jax.ShapeDtypeStruct((_NC, _G, _D), jnp.float32),
    scratch_types=[
        pltpu.VMEM((_ROUNDS, _CHUNK), jnp.int32),
        pltpu.VMEM((2, _CHUNK, _D), jnp.float32),
        pltpu.VMEM((_ZROWS, _D), jnp.float32),
        pltpu.VMEM_SHARED((_ACC_ROWS, _D), jnp.float32),
        pltpu.SemaphoreType.DMA,
        pltpu.SemaphoreType.DMA,
        pltpu.SemaphoreType.DMA,
        pltpu.SemaphoreType.DMA,
        pltpu.SemaphoreType.DMA,
    ],
)
def _sc_segment_sum(x_hbm, b_hbm, out_hbm, idx_v, rows_v, zbuf, acc_sh,
                    isem, fs0, fs1, ss0, ss1):
    c = lax.axis_index("c")
    s = lax.axis_index("s")
    w = c * _NS + s
    fsem = (fs0, fs1)
    ssem = (ss0, ss1)

    def _chunk_off(j):
        return (j * _NW + w) * _CHUNK

    def _fetch(j):
        b = j % 2
        return pltpu.async_copy(x_hbm.at[pl.ds(_chunk_off(j), _CHUNK)],
                                rows_v.at[b], fsem[b])

    idx_h = [
        pltpu.async_copy(b_hbm.at[pl.ds(_chunk_off(j), _CHUNK)],
                         idx_v.at[j], isem)
        for j in range(_JFULL)
    ]

    zero = jnp.zeros((16,), jnp.float32)
    for r in range(_ZROWS):
        for k in range(_D // 16):
            zbuf[r, pl.ds(k * 16, 16)] = zero
    pltpu.sync_copy(zbuf, acc_sh.at[pl.ds(s * _ZROWS, _ZROWS)])
    plsc.subcore_barrier()

    fh = [_fetch(0), _fetch(1)]
    sh = [None, None]
    for j in range(_JFULL):
        b = j % 2
        fh[b].wait()
        idx_h[j].wait()
        sh[b] = pltpu.async_copy(rows_v.at[b], acc_sh.at[idx_v.at[j]],
                                 ssem[b], add=True)
        if j + 2 < _JFULL:
            sh[b].wait()
            fh[b] = _fetch(j + 2)
    sh[0].wait()
    sh[1].wait()

    @pl.when(w < _REM)
    def _leftover():
        off = _chunk_off(_JFULL)
        pltpu.sync_copy(b_hbm.at[pl.ds(off, _CHUNK)], idx_v.at[_JFULL])
        pltpu.sync_copy(x_hbm.at[pl.ds(off, _CHUNK)], rows_v.at[0])
        pltpu.sync_copy(rows_v.at[0], acc_sh.at[idx_v.at[_JFULL]], add=True)

    @pl.when(w == _REM)
    def _tail():
        off = _FULL * _CHUNK
        dummy = jnp.full((16,), _DUMMY, jnp.int32)
        for k in range(_TAIL // 16, _CHUNK // 16):
            idx_v[_JFULL, pl.ds(k * 16, 16)] = dummy
        pltpu.sync_copy(b_hbm.at[pl.ds(off, _TAIL)],
                        idx_v.at[_JFULL, pl.ds(0, _TAIL)])
        pltpu.sync_copy(x_hbm.at[pl.ds(off, _TAIL)],
                        rows_v.at[0, pl.ds(0, _TAIL)])
        pltpu.sync_copy(rows_v.at[0], acc_sh.at[idx_v.at[_JFULL]], add=True)

    plsc.subcore_barrier()

    pltpu.sync_copy(acc_sh.at[pl.ds(s * 32, 32)], zbuf.at[pl.ds(0, 32)])
    pltpu.sync_copy(zbuf.at[pl.ds(0, 32)], out_hbm.at[c, pl.ds(s * 32, 32)])


def _tc_combine(bias_ref, p_ref, o_ref):
    o_ref[...] = p_ref[0] + p_ref[1] + bias_ref[0]


def kernel(input, batch, num_graphs):
    x = input.astype(jnp.float32)
    b = batch.astype(jnp.int32)
    partials = _sc_segment_sum(x, b)
    bias = (jnp.asarray(num_graphs) - _G).astype(jnp.float32).reshape(1)
    out = pl.pallas_call(
        _tc_combine,
        out_shape=jax.ShapeDtypeStruct((_G, _D), jnp.float32),
        in_specs=[
            pl.BlockSpec(memory_space=pltpu.SMEM),
            pl.BlockSpec(memory_space=pltpu.VMEM),
        ],
    )(bias, partials)
    return out


# speedup vs baseline: 6.7222x; 1.0221x over previous
import functools

import jax
import jax.numpy as jnp
from jax import lax
from jax.experimental import pallas as pl
from jax.experimental.pallas import tpu as pltpu
from jax.experimental.pallas import tpu_sc as plsc

_N = 100000
_D = 128
_G = 512
_NC = 2
_NS = 16
_NW = _NC * _NS
_CHUNK = 128
_FULL = _N // _CHUNK
_TAIL = _N - _FULL * _CHUNK
_JFULL = _FULL // _NW
_REM = _FULL - _JFULL * _NW
_ROUNDS = _JFULL + 1
_ACC_ROWS = 528
_ZROWS = _ACC_ROWS // _NS
_DUMMY = _G

_mesh = plsc.VectorSubcoreMesh(core_axis_name="c", subcore_axis_name="s")


@functools.partial(
    pl.kernel,
    mesh=_mesh,
    out_type=jax.ShapeDtypeStruct((_NC, _G, _D), jnp.float32),
    scratch_types=[
        pltpu.VMEM((_ROUNDS, _CHUNK), jnp.int32),
        pltpu.VMEM((4, _CHUNK, _D), jnp.float32),
        pltpu.VMEM((_ZROWS, _D), jnp.float32),
        pltpu.VMEM_SHARED((_ACC_ROWS, _D), jnp.float32),
        pltpu.SemaphoreType.DMA,
        pltpu.SemaphoreType.DMA,
        pltpu.SemaphoreType.DMA,
        pltpu.SemaphoreType.DMA,
        pltpu.SemaphoreType.DMA,
        pltpu.SemaphoreType.DMA,
        pltpu.SemaphoreType.DMA,
        pltpu.SemaphoreType.DMA,
        pltpu.SemaphoreType.DMA,
    ],
)
def _sc_segment_sum(x_hbm, b_hbm, out_hbm, idx_v, rows_v, zbuf, acc_sh,
                    isem, fs0, fs1, fs2, fs3, ss0, ss1, ss2, ss3):
    c = lax.axis_index("c")
    s = lax.axis_index("s")
    w = c * _NS + s
    fsem = (fs0, fs1, fs2, fs3)
    ssem = (ss0, ss1, ss2, ss3)

    def _chunk_off(j):
        return (j * _NW + w) * _CHUNK

    def _fetch(j):
        b = j % 4
        return pltpu.async_copy(x_hbm.at[pl.ds(_chunk_off(j), _CHUNK)],
                                rows_v.at[b], fsem[b])

    idx_h = [
        pltpu.async_copy(b_hbm.at[pl.ds(_chunk_off(j), _CHUNK)],
                         idx_v.at[j], isem)
        for j in range(_JFULL)
    ]

    zero = jnp.zeros((16,), jnp.float32)
    for r in range(_ZROWS):
        for k in range(_D // 16):
            zbuf[r, pl.ds(k * 16, 16)] = zero
    pltpu.sync_copy(zbuf, acc_sh.at[pl.ds(s * _ZROWS, _ZROWS)])
    plsc.subcore_barrier()

    fh = [_fetch(0), _fetch(1), _fetch(2), None]
    sh = [None, None, None, None]
    for j in range(_JFULL):
        b = j % 4
        fh[b].wait()
        idx_h[j].wait()
        sh[b] = pltpu.async_copy(rows_v.at[b], acc_sh.at[idx_v.at[j]],
                                 ssem[b], add=True)
        nj = j + 3
        if nj < _JFULL:
            nb = nj % 4
            if sh[nb] is not None:
                sh[nb].wait()
                sh[nb] = None
            fh[nb] = _fetch(nj)
    for b in range(4):
        if sh[b] is not None:
            sh[b].wait()

    @pl.when(w < _REM)
    def _leftover():
        off = _chunk_off(_JFULL)
        pltpu.sync_copy(b_hbm.at[pl.ds(off, _CHUNK)], idx_v.at[_JFULL])
        pltpu.sync_copy(x_hbm.at[pl.ds(off, _CHUNK)], rows_v.at[0])
        pltpu.sync_copy(rows_v.at[0], acc_sh.at[idx_v.at[_JFULL]], add=True)

    @pl.when(w == _REM)
    def _tail():
        off = _FULL * _CHUNK
        dummy = jnp.full((16,), _DUMMY, jnp.int32)
        for k in range(_TAIL // 16, _CHUNK // 16):
            idx_v[_JFULL, pl.ds(k * 16, 16)] = dummy
        pltpu.sync_copy(b_hbm.at[pl.ds(off, _TAIL)],
                        idx_v.at[_JFULL, pl.ds(0, _TAIL)])
        pltpu.sync_copy(x_hbm.at[pl.ds(off, _TAIL)],
                        rows_v.at[0, pl.ds(0, _TAIL)])
        pltpu.sync_copy(rows_v.at[0], acc_sh.at[idx_v.at[_JFULL]], add=True)

    plsc.subcore_barrier()

    pltpu.sync_copy(acc_sh.at[pl.ds(s * 32, 32)], zbuf.at[pl.ds(0, 32)])
    pltpu.sync_copy(zbuf.at[pl.ds(0, 32)], out_hbm.at[c, pl.ds(s * 32, 32)])


def _tc_combine(bias_ref, p_ref, o_ref):
    o_ref[...] = p_ref[0] + p_ref[1] + bias_ref[0]


def kernel(input, batch, num_graphs):
    x = input.astype(jnp.float32)
    b = batch.astype(jnp.int32)
    partials = _sc_segment_sum(x, b)
    bias = (jnp.asarray(num_graphs) - _G).astype(jnp.float32).reshape(1)
    out = pl.pallas_call(
        _tc_combine,
        out_shape=jax.ShapeDtypeStruct((_G, _D), jnp.float32),
        in_specs=[
            pl.BlockSpec(memory_space=pltpu.SMEM),
            pl.BlockSpec(memory_space=pltpu.VMEM),
        ],
    )(bias, partials)
    return out


# speedup vs baseline: 6.8967x; 1.0260x over previous
import functools

import jax
import jax.numpy as jnp
from jax import lax
from jax.experimental import pallas as pl
from jax.experimental.pallas import tpu as pltpu
from jax.experimental.pallas import tpu_sc as plsc

_N = 100000
_D = 128
_G = 512
_NC = 2
_NS = 16
_NW = _NC * _NS
_CHUNK = 128
_FULL = _N // _CHUNK
_TAIL = _N - _FULL * _CHUNK
_JFULL = _FULL // _NW
_REM = _FULL - _JFULL * _NW
_ROUNDS = _JFULL + 1
_ACC_ROWS = 528
_ZROWS = _ACC_ROWS // _NS
_DUMMY = _G

_mesh = plsc.VectorSubcoreMesh(core_axis_name="c", subcore_axis_name="s")


@functools.partial(
    pl.kernel,
    mesh=_mesh,
    out_type=jax.ShapeDtypeStruct((_NC, _G, _D), jnp.float32),
    scratch_types=[
        pltpu.VMEM((_ROUNDS, _CHUNK), jnp.int32),
        pltpu.VMEM((5, _CHUNK, _D), jnp.float32),
        pltpu.VMEM((_ZROWS, _D), jnp.float32),
        pltpu.VMEM_SHARED((_ACC_ROWS, _D), jnp.float32),
        pltpu.SemaphoreType.DMA,
        pltpu.SemaphoreType.DMA,
        pltpu.SemaphoreType.DMA,
        pltpu.SemaphoreType.DMA,
        pltpu.SemaphoreType.DMA,
        pltpu.SemaphoreType.DMA,
        pltpu.SemaphoreType.DMA,
        pltpu.SemaphoreType.DMA,
        pltpu.SemaphoreType.DMA,
        pltpu.SemaphoreType.DMA,
    ],
)
def _sc_segment_sum(x_hbm, b_hbm, out_hbm, idx_v, rows_v, zbuf, acc_sh,
                    isem, fs0, fs1, fs2, fs3, ss0, ss1, ss2, ss3, fs4):
    c = lax.axis_index("c")
    s = lax.axis_index("s")
    w = c * _NS + s
    fsem = (fs0, fs1, fs2, fs3)
    ssem = (ss0, ss1, ss2, ss3)

    def _chunk_off(j):
        return (j * _NW + w) * _CHUNK

    def _fetch(j):
        b = j % 4
        return pltpu.async_copy(x_hbm.at[pl.ds(_chunk_off(j), _CHUNK)],
                                rows_v.at[b], fsem[b])

    fh = [_fetch(0), _fetch(1), _fetch(2), None]

    idx_h = [
        pltpu.async_copy(b_hbm.at[pl.ds(_chunk_off(j), _CHUNK)],
                         idx_v.at[j], isem)
        for j in range(_JFULL)
    ]

    @pl.when(w < _REM)
    def _leftover_start():
        off = _chunk_off(_JFULL)
        pltpu.async_copy(b_hbm.at[pl.ds(off, _CHUNK)], idx_v.at[_JFULL], isem)
        pltpu.async_copy(x_hbm.at[pl.ds(off, _CHUNK)], rows_v.at[4], fs4)

    @pl.when(w == _REM)
    def _tail_start():
        off = _FULL * _CHUNK
        pltpu.async_copy(b_hbm.at[pl.ds(off, _TAIL)],
                         idx_v.at[_JFULL, pl.ds(0, _TAIL)], isem)
        pltpu.async_copy(x_hbm.at[pl.ds(off, _TAIL)],
                         rows_v.at[4, pl.ds(0, _TAIL)], fs4)

    zero = jnp.zeros((16,), jnp.float32)
    for r in range(_ZROWS):
        for k in range(_D // 16):
            zbuf[r, pl.ds(k * 16, 16)] = zero
    pltpu.sync_copy(zbuf, acc_sh.at[pl.ds(s * _ZROWS, _ZROWS)])
    plsc.subcore_barrier()

    sh = [None, None, None, None]
    for j in range(_JFULL):
        b = j % 4
        fh[b].wait()
        idx_h[j].wait()
        sh[b] = pltpu.async_copy(rows_v.at[b], acc_sh.at[idx_v.at[j]],
                                 ssem[b], add=True)
        nj = j + 3
        if nj < _JFULL:
            nb = nj % 4
            if sh[nb] is not None:
                sh[nb].wait()
                sh[nb] = None
            fh[nb] = _fetch(nj)
    for b in range(4):
        if sh[b] is not None:
            sh[b].wait()

    @pl.when(w < _REM)
    def _leftover_finish():
        off = _chunk_off(_JFULL)
        pltpu.make_async_copy(b_hbm.at[pl.ds(off, _CHUNK)],
                              idx_v.at[_JFULL], isem).wait()
        pltpu.make_async_copy(x_hbm.at[pl.ds(off, _CHUNK)],
                              rows_v.at[4], fs4).wait()
        pltpu.sync_copy(rows_v.at[4], acc_sh.at[idx_v.at[_JFULL]], add=True)

    @pl.when(w == _REM)
    def _tail_finish():
        off = _FULL * _CHUNK
        dummy = jnp.full((16,), _DUMMY, jnp.int32)
        for k in range(_TAIL // 16, _CHUNK // 16):
            idx_v[_JFULL, pl.ds(k * 16, 16)] = dummy
        pltpu.make_async_copy(b_hbm.at[pl.ds(off, _TAIL)],
                              idx_v.at[_JFULL, pl.ds(0, _TAIL)], isem).wait()
        pltpu.make_async_copy(x_hbm.at[pl.ds(off, _TAIL)],
                              rows_v.at[4, pl.ds(0, _TAIL)], fs4).wait()
        pltpu.sync_copy(rows_v.at[4], acc_sh.at[idx_v.at[_JFULL]], add=True)

    plsc.subcore_barrier()

    pltpu.sync_copy(acc_sh.at[pl.ds(s * 32, 32)], out_hbm.at[c, pl.ds(s * 32, 32)])


def _tc_combine(bias_ref, p_ref, o_ref):
    o_ref[...] = p_ref[0] + p_ref[1] + bias_ref[0]


def kernel(input, batch, num_graphs):
    x = input.astype(jnp.float32)
    b = batch.astype(jnp.int32)
    partials = _sc_segment_sum(x, b)
    bias = (jnp.asarray(num_graphs) - _G).astype(jnp.float32).reshape(1)
    out = pl.pallas_call(
        _tc_combine,
        out_shape=jax.ShapeDtypeStruct((_G, _D), jnp.float32),
        in_specs=[
            pl.BlockSpec(memory_space=pltpu.SMEM),
            pl.BlockSpec(memory_space=pltpu.VMEM),
        ],
    )(bias, partials)
    return out
